# baseline (speedup 1.0000x reference)
import dataclasses

import jax
import jax.numpy as jnp
from jax import lax
from jax.experimental import pallas as pl
from jax.experimental.pallas import tpu as pltpu
from jax.experimental.pallas import tpu_sc as plsc

BATCH = 4
SEQ = 2048
HIDDEN = 1024
EPS = 1e-12

L = 16
NC, NS = 2, 16
NW = NC * NS
POS_PER_W = SEQ // NW
K = 16
NCHUNK = POS_PER_W // K
NSLICE = HIDDEN // L


def _rsqrt16(x):
    i = lax.bitcast_convert_type(x, jnp.int32)
    i = jnp.int32(0x5F3759DF) - lax.shift_right_logical(i, 1)
    y = lax.bitcast_convert_type(i, jnp.float32)
    for _ in range(3):
        y = y * (1.5 - 0.5 * x * y * y)
    return y


def _body(tok_ids, type_ids, tok_tab, pos_tab, type_tab, gamma, beta, out,
          idx_v, tt_v, tok_v, pos_v, out_v, dif_v, g_v, b_v, ttab_v, sem):
    wid = lax.axis_index("s") * NC + lax.axis_index("c")
    pbase = wid * POS_PER_W

    pltpu.sync_copy(type_tab, ttab_v)
    pltpu.sync_copy(gamma, g_v)
    pltpu.sync_copy(beta, b_v)

    @pl.loop(0, NSLICE)
    def _pre(s):
        sl = pl.ds(s * L, L)
        dif_v[sl] = ttab_v[1, sl] - ttab_v[0, sl]

    @pl.loop(0, NCHUNK)
    def _chunk(ci):
        p0 = pbase + ci * K
        pltpu.sync_copy(pos_tab.at[pl.ds(p0, K)], pos_v)

        @pl.loop(0, BATCH)
        def _batch(b):
            pltpu.sync_copy(tok_ids.at[b, pl.ds(p0, K)], idx_v)
            pltpu.sync_copy(type_ids.at[b, pl.ds(p0, K)], tt_v)
            pltpu.async_copy(tok_tab.at[idx_v], tok_v, sem).wait()

            @pl.loop(0, K)
            def _row(r):
                tvec = plsc.load_gather(tt_v, [jnp.full((L,), r, jnp.int32)])
                tf = tvec.astype(jnp.float32)

                def p1(s, carry):
                    acc, acq = carry
                    sl = pl.ds(s * L, L)
                    x = tok_v[r, sl] + pos_v[r, sl] + (ttab_v[0, sl] + tf * dif_v[sl])
                    tok_v[r, sl] = x
                    return acc + x, acq + x * x

                acc, acq = lax.fori_loop(
                    0, NSLICE, p1,
                    (jnp.zeros((L,), jnp.float32), jnp.zeros((L,), jnp.float32)))
                mean = jnp.full((L,), jnp.sum(acc)) * (1.0 / HIDDEN)
                var = jnp.full((L,), jnp.sum(acq)) * (1.0 / HIDDEN) - mean * mean
                rstd = _rsqrt16(var + EPS)

                def p2(s, _):
                    sl = pl.ds(s * L, L)
                    x = tok_v[r, sl]
                    out_v[r, sl] = (x - mean) * rstd * g_v[sl] + b_v[sl]
                    return 0

                lax.fori_loop(0, NSLICE, p2, 0)

            pltpu.sync_copy(out_v, out.at[b, pl.ds(p0, K)])


def kernel(token_ids, token_type_ids, tok_table, pos_table, type_table, gamma, beta):
    mesh = plsc.VectorSubcoreMesh(core_axis_name="c", subcore_axis_name="s")
    cp = pltpu.CompilerParams()
    if "needs_layout_passes" in pltpu.CompilerParams.__dataclass_fields__:
        cp = dataclasses.replace(cp, needs_layout_passes=False)
    run = pl.kernel(
        _body,
        out_type=jax.ShapeDtypeStruct((BATCH, SEQ, HIDDEN), jnp.float32),
        mesh=mesh,
        scratch_types=[
            pltpu.VMEM((K,), jnp.int32),
            pltpu.VMEM((K,), jnp.int32),
            pltpu.VMEM((K, HIDDEN), jnp.float32),
            pltpu.VMEM((K, HIDDEN), jnp.float32),
            pltpu.VMEM((K, HIDDEN), jnp.float32),
            pltpu.VMEM((HIDDEN,), jnp.float32),
            pltpu.VMEM((HIDDEN,), jnp.float32),
            pltpu.VMEM((HIDDEN,), jnp.float32),
            pltpu.VMEM((2, HIDDEN), jnp.float32),
            pltpu.SemaphoreType.DMA,
        ],
        compiler_params=cp,
    )
    return run(token_ids.astype(jnp.int32), token_type_ids.astype(jnp.int32),
               tok_table, pos_table, type_table, gamma, beta)
